# baseline (speedup 1.0000x reference)
import functools

import numpy as np
import jax
import jax.numpy as jnp
from jax import lax
from jax.experimental import pallas as pl
from jax.experimental.pallas import tpu as pltpu
from jax.experimental.pallas import tpu_sc as plsc

F32 = jnp.float32
BF16 = jnp.bfloat16
H = 128
NSTEP = 50
EPS = 1e-5

SC_CORES = 2
SC_SUBCORES = 16
SC_WORKERS = SC_CORES * SC_SUBCORES


def _lrelu(x):
    return jnp.where(x >= 0, x, 0.01 * x)


_MASK_HI = -65536
_MASK_LO = 0xFFFF


def _rne_bf16_bits(f):
    b = lax.bitcast_convert_type(f, jnp.int32)
    return b + 0x7FFF + ((b >> 16) & 1)


def _pack_pair(lo, hi):
    return (((_rne_bf16_bits(lo) >> 16) & _MASK_LO)
            | (_rne_bf16_bits(hi) & _MASK_HI))


def _unpack_lo(p):
    return lax.bitcast_convert_type(p << 16, F32)


def _unpack_hi(p):
    return lax.bitcast_convert_type(p & _MASK_HI, F32)


def _dot16(a, b):
    return jnp.dot(a.astype(BF16), b.astype(BF16), preferred_element_type=F32)


def _prep_body(x_ref, w1a_ref, w1c_ref, wd_ref, bd_ref, tsrc_ref, u3_ref,
               xd_ref):
    xb = x_ref[...]
    xd = jnp.dot(xb, wd_ref[...], preferred_element_type=F32) + bd_ref[...]
    u1 = jnp.dot(xb, w1a_ref[...], preferred_element_type=F32)
    tsrc_ref[...] = _pack_pair(u1, xd)
    u3_ref[...] = jnp.dot(xb, w1c_ref[...], preferred_element_type=F32)
    xd_ref[...] = xd


def _prep(x, w1a, w1c, wd, bd, nblk):
    n = x.shape[0]
    grid = (n // nblk,)
    return pl.pallas_call(
        _prep_body,
        grid=grid,
        in_specs=[
            pl.BlockSpec((nblk, H), lambda i: (i, 0)),
            pl.BlockSpec((H, H), lambda i: (0, 0)),
            pl.BlockSpec((H, H), lambda i: (0, 0)),
            pl.BlockSpec((H, H), lambda i: (0, 0)),
            pl.BlockSpec((1, H), lambda i: (0, 0)),
        ],
        out_specs=[
            pl.BlockSpec((nblk, H), lambda i: (i, 0)),
            pl.BlockSpec((nblk, H), lambda i: (i, 0)),
            pl.BlockSpec((nblk, H), lambda i: (i, 0)),
        ],
        out_shape=[
            jax.ShapeDtypeStruct((n, H), jnp.int32),
            jax.ShapeDtypeStruct((n, H), F32),
            jax.ShapeDtypeStruct((n, H), F32),
        ],
    )(x, w1a, w1c, wd, bd)


def _sc_gather(tsrc, u3, ei, e0, e):
    tc = e // 128
    cw = tc // SC_WORKERS
    rem = tc - cw * SC_WORKERS
    assert e % 128 == 0 and rem % 2 == 0
    stage = (cw + rem) * 128
    mesh = plsc.VectorSubcoreMesh(core_axis_name="c", subcore_axis_name="s")

    @functools.partial(
        pl.kernel,
        mesh=mesh,
        out_type=[
            jax.ShapeDtypeStruct((e, H), jnp.int32),
            jax.ShapeDtypeStruct((e, H), F32),
        ],
        scratch_types=[
            pltpu.VMEM((stage,), jnp.int32),
            pltpu.VMEM((stage,), jnp.int32),
            pltpu.VMEM((128, H), jnp.int32),
            pltpu.VMEM((128, H), jnp.int32),
            pltpu.VMEM((128, H), F32),
            pltpu.VMEM((128, H), F32),
            pltpu.SemaphoreType.DMA,
            pltpu.SemaphoreType.DMA,
            pltpu.SemaphoreType.DMA,
            pltpu.SemaphoreType.DMA,
        ],
    )
    def k(tsrc_hbm, u3_hbm, ei_hbm, gs_hbm, gd_hbm,
          idxs, idxd, rs0, rs1, rd0, rd1, sem_s0, sem_s1, sem_d0, sem_d1):
        c = lax.axis_index("c")
        s = lax.axis_index("s")
        wid = s * SC_CORES + c
        base = wid * cw * 128
        nf = cw + jnp.where(wid == SC_WORKERS - 1, rem, 0)

        pltpu.sync_copy(ei_hbm.at[0, pl.ds(e0 + base, stage)], idxs)
        pltpu.sync_copy(ei_hbm.at[1, pl.ds(e0 + base, stage)], idxd)

        def start(j, rbuf_s, rbuf_d, sem_a, sem_b):
            pltpu.async_copy(tsrc_hbm.at[idxs.at[pl.ds(j * 128, 128)]], rbuf_s, sem_a)
            pltpu.async_copy(u3_hbm.at[idxd.at[pl.ds(j * 128, 128)]], rbuf_d, sem_b)

        def finish(j, rbuf_s, rbuf_d, sem_a, sem_b):
            pltpu.make_async_copy(tsrc_hbm.at[idxs.at[pl.ds(j * 128, 128)]], rbuf_s, sem_a).wait()
            pltpu.make_async_copy(u3_hbm.at[idxd.at[pl.ds(j * 128, 128)]], rbuf_d, sem_b).wait()
            pltpu.sync_copy(rbuf_s, gs_hbm.at[pl.ds(base + j * 128, 128)])
            pltpu.sync_copy(rbuf_d, gd_hbm.at[pl.ds(base + j * 128, 128)])

        start(0, rs0, rd0, sem_s0, sem_d0)

        def body(j2, carry):
            j = j2 * 2
            start(j + 1, rs1, rd1, sem_s1, sem_d1)
            finish(j, rs0, rd0, sem_s0, sem_d0)

            @pl.when(j + 2 < nf)
            def _():
                start(j + 2, rs0, rd0, sem_s0, sem_d0)

            finish(j + 1, rs1, rd1, sem_s1, sem_d1)
            return carry

        lax.fori_loop(0, nf // 2, body, 0)
        if cw % 2:
            finish(nf - 1, rs0, rd0, sem_s0, sem_d0)

    return k(tsrc, u3, ei)


_DN0 = (((0,), (0,)), ((), ()))


def _pass1_body(ea_ref, gs_ref, gd_ref, r_ref, cf_ref, wb_ref, bb_ref,
                wmid_ref, be1_ref, h1_ref, st_ref, acc):
    i = pl.program_id(0)
    a = lax.dot_general(ea_ref[...], r_ref[...], _DN0,
                        preferred_element_type=F32)
    d = a - cf_ref[...]
    g = jnp.exp(-(d * d))
    g = _lrelu(_dot16(g, wb_ref[...]) + bb_ref[...])
    h1 = _lrelu(_unpack_lo(gs_ref[...]) + gd_ref[...] +
                _dot16(g, wmid_ref[...]) + be1_ref[...])
    h1_ref[...] = h1.astype(BF16)

    @pl.when(i == 0)
    def _():
        acc[...] = jnp.zeros_like(acc)

    acc[0:1, :] += jnp.sum(h1, axis=0, keepdims=True)
    acc[1:2, :] += jnp.sum(h1 * h1, axis=0, keepdims=True)

    @pl.when(i == pl.num_programs(0) - 1)
    def _():
        st_ref[...] = acc[...]


def _pass1(ea, gs, gd, rmat, cf, wb, bb, wmid, be1, eb, blk0):
    e = gs.shape[0]
    grid = (e // eb,)
    return pl.pallas_call(
        _pass1_body,
        grid=grid,
        in_specs=[
            pl.BlockSpec((4, eb), lambda i: (0, blk0 + i)),
            pl.BlockSpec((eb, H), lambda i: (i, 0)),
            pl.BlockSpec((eb, H), lambda i: (i, 0)),
            pl.BlockSpec((4, 4 * NSTEP), lambda i: (0, 0)),
            pl.BlockSpec((1, 4 * NSTEP), lambda i: (0, 0)),
            pl.BlockSpec((4 * NSTEP, H), lambda i: (0, 0)),
            pl.BlockSpec((1, H), lambda i: (0, 0)),
            pl.BlockSpec((H, H), lambda i: (0, 0)),
            pl.BlockSpec((1, H), lambda i: (0, 0)),
        ],
        out_specs=[
            pl.BlockSpec((eb, H), lambda i: (i, 0)),
            pl.BlockSpec((8, H), lambda i: (0, 0)),
        ],
        out_shape=[
            jax.ShapeDtypeStruct((e, H), BF16),
            jax.ShapeDtypeStruct((8, H), F32),
        ],
        scratch_shapes=[pltpu.VMEM((8, H), F32)],
    )(ea, gs, gd, rmat, cf, wb, bb, wmid, be1)


def _bn_fold(st, ne, g, bt, w_next, b_next):
    mean = st[0] * (1.0 / ne)
    var = st[1] * (1.0 / ne) - mean * mean
    a = g * lax.rsqrt(var + EPS)
    weff = w_next * a[:, None]
    beff = ((bt - mean * a) @ w_next + b_next).reshape(1, -1)
    return weff, beff


def _stats2_body(h1_ref, w2_ref, b2_ref, st2_ref, acc):
    i = pl.program_id(0)
    h2 = _lrelu(_dot16(h1_ref[...], w2_ref[...]) + b2_ref[...])

    @pl.when(i == 0)
    def _():
        acc[...] = jnp.zeros_like(acc)

    acc[0:1, :] += jnp.sum(h2, axis=0, keepdims=True)
    acc[1:2, :] += jnp.sum(h2 * h2, axis=0, keepdims=True)

    @pl.when(i == pl.num_programs(0) - 1)
    def _():
        st2_ref[...] = acc[...]


def _stats2(h1, w2eff, b2eff, eb):
    e = h1.shape[0]
    grid = (e // eb,)
    return pl.pallas_call(
        _stats2_body,
        grid=grid,
        in_specs=[
            pl.BlockSpec((eb, H), lambda i: (i, 0)),
            pl.BlockSpec((H, H), lambda i: (0, 0)),
            pl.BlockSpec((1, H), lambda i: (0, 0)),
        ],
        out_specs=pl.BlockSpec((8, H), lambda i: (0, 0)),
        out_shape=jax.ShapeDtypeStruct((8, H), F32),
        scratch_shapes=[pltpu.VMEM((8, H), F32)],
    )(h1, w2eff, b2eff)


def _msgs_body(h1_ref, gxd_ref, ea_ref, one_ref, w2_ref, b2_ref,
               w3_ref, b3_ref, m_ref):
    h2 = _lrelu(_dot16(h1_ref[...], w2_ref[...]) + b2_ref[...])
    h3 = _dot16(h2, w3_ref[...]) + b3_ref[...]
    cosr = jnp.cos((np.pi / 2) * ea_ref[3:4, :])
    coef = lax.dot_general(cosr, one_ref[...], _DN0,
                           preferred_element_type=F32)
    m_ref[...] = coef[:, 0:1] * h3 * _unpack_hi(gxd_ref[...])


def _msgs(h1, gs, eat, one18, w2eff, b2eff, w3eff, b3eff, eb, blk0):
    e = h1.shape[0]
    grid = (e // eb,)
    return pl.pallas_call(
        _msgs_body,
        grid=grid,
        in_specs=[
            pl.BlockSpec((eb, H), lambda i: (i, 0)),
            pl.BlockSpec((eb, H), lambda i: (i, 0)),
            pl.BlockSpec((4, eb), lambda i: (0, blk0 + i)),
            pl.BlockSpec((1, 8), lambda i: (0, 0)),
            pl.BlockSpec((H, H), lambda i: (0, 0)),
            pl.BlockSpec((1, H), lambda i: (0, 0)),
            pl.BlockSpec((H, H), lambda i: (0, 0)),
            pl.BlockSpec((1, H), lambda i: (0, 0)),
        ],
        out_specs=pl.BlockSpec((eb, H), lambda i: (i, 0)),
        out_shape=jax.ShapeDtypeStruct((e, H), F32),
    )(h1, gs, eat, one18, w2eff, b2eff, w3eff, b3eff)


def _sc_scatter(m, ei, init, e0):
    e = m.shape[0]
    n = init.shape[1]
    tc = e // 128
    cw = tc // SC_WORKERS
    rem = tc - cw * SC_WORKERS
    assert e % 128 == 0 and rem % 2 == 0
    mesh = plsc.VectorSubcoreMesh(core_axis_name="c", subcore_axis_name="s")

    @functools.partial(
        pl.kernel,
        mesh=mesh,
        out_type=jax.ShapeDtypeStruct((SC_CORES, n, H), F32),
        scratch_types=[
            pltpu.VMEM((128,), jnp.int32),
            pltpu.VMEM((128,), jnp.int32),
            pltpu.VMEM((16,), jnp.int32),
            pltpu.VMEM((128, H), F32),
            pltpu.VMEM((128, H), F32),
            pltpu.VMEM_SHARED((n, H), F32),
            pltpu.SemaphoreType.DMA,
            pltpu.SemaphoreType.DMA,
            pltpu.SemaphoreType.DMA,
            pltpu.SemaphoreType.DMA,
        ],
    )
    def k(m_hbm, ei_hbm, z_hbm, out_hbm, idx0, idx1, idxt, rb0, rb1, accsh,
          si0, si1, sm0, sm1):
        c = lax.axis_index("c")
        s = lax.axis_index("s")
        r0 = s * 640
        ncp = jnp.where(s == SC_SUBCORES - 1, (n - 640 * (SC_SUBCORES - 1)) // 40,
                        640 // 40)

        def cp_init(j, carry):
            off = r0 + j * 40
            pltpu.sync_copy(z_hbm.at[c, pl.ds(off, 40)], accsh.at[pl.ds(off, 40)])
            return carry

        lax.fori_loop(0, ncp, cp_init, 0)
        wid = c * SC_SUBCORES + s
        base = wid * cw * 128
        nf = cw + jnp.where(wid == SC_WORKERS - 1, rem, 0)
        plsc.subcore_barrier()

        def start(j, idxb, rbuf, semi, semm):
            pltpu.async_copy(ei_hbm.at[1, pl.ds(e0 + base + j * 128, 128)], idxb, semi)
            pltpu.async_copy(m_hbm.at[pl.ds(base + j * 128, 128)], rbuf, semm)

        def finish(j, idxb, rbuf, semi, semm):
            pltpu.make_async_copy(ei_hbm.at[1, pl.ds(e0 + base + j * 128, 128)], idxb, semi).wait()
            pltpu.make_async_copy(m_hbm.at[pl.ds(base + j * 128, 128)], rbuf, semm).wait()
            pltpu.sync_copy(rbuf, accsh.at[idxb], add=True)

        start(0, idx0, rb0, si0, sm0)

        def body(j2, carry):
            j = j2 * 2
            start(j + 1, idx1, rb1, si1, sm1)
            finish(j, idx0, rb0, si0, sm0)

            @pl.when(j + 2 < nf)
            def _():
                start(j + 2, idx0, rb0, si0, sm0)

            finish(j + 1, idx1, rb1, si1, sm1)
            return carry

        lax.fori_loop(0, nf // 2, body, 0)
        if cw % 2:
            finish(nf - 1, idx0, rb0, si0, sm0)
        plsc.subcore_barrier()

        def cp_out(j, carry):
            off = r0 + j * 40
            pltpu.sync_copy(accsh.at[pl.ds(off, 40)], out_hbm.at[c, pl.ds(off, 40)])
            return carry

        lax.fori_loop(0, ncp, cp_out, 0)

    return k(m, ei, init)


def _node_stats_body(xd_ref, i0_ref, i1_ref, v_ref, wn1_ref, bn1_ref,
                     st_ref, acc):
    i = pl.program_id(0)
    z0 = v_ref[...] * xd_ref[...] + i0_ref[0] + i1_ref[0]
    n1 = _lrelu(jnp.dot(z0, wn1_ref[...], preferred_element_type=F32) + bn1_ref[...])

    @pl.when(i == 0)
    def _():
        acc[...] = jnp.zeros_like(acc)

    acc[0:1, :] += jnp.sum(n1, axis=0, keepdims=True)
    acc[1:2, :] += jnp.sum(n1 * n1, axis=0, keepdims=True)

    @pl.when(i == pl.num_programs(0) - 1)
    def _():
        st_ref[...] = acc[...]


def _node_apply_body(xd_ref, i0_ref, i1_ref, x_ref, v_ref, wn1_ref,
                     bn1_ref, wn2_ref, bn2_ref, z_ref):
    z0 = v_ref[...] * xd_ref[...] + i0_ref[0] + i1_ref[0]
    n1 = _lrelu(jnp.dot(z0, wn1_ref[...], preferred_element_type=F32) + bn1_ref[...])
    z_ref[...] = (jnp.dot(n1, wn2_ref[...], preferred_element_type=F32)
                  + bn2_ref[...] + x_ref[...])


def _node_stats(xd, inc, v, wn1, bn1, nblk):
    n = xd.shape[0]
    grid = (n // nblk,)
    blk = lambda shape: pl.BlockSpec(shape, lambda i: (0,) * len(shape))
    nodeb = pl.BlockSpec((nblk, H), lambda i: (i, 0))
    inc0 = pl.BlockSpec((1, nblk, H), lambda i: (0, i, 0))
    inc1 = pl.BlockSpec((1, nblk, H), lambda i: (1, i, 0))
    return pl.pallas_call(
        _node_stats_body,
        grid=grid,
        in_specs=[nodeb, inc0, inc1, blk((1, H)), blk((H, H)), blk((1, H))],
        out_specs=pl.BlockSpec((8, H), lambda i: (0, 0)),
        out_shape=jax.ShapeDtypeStruct((8, H), F32),
        scratch_shapes=[pltpu.VMEM((8, H), F32)],
    )(xd, inc, inc, v, wn1, bn1)


def _node_apply(xd, inc, x, v, wn1, bn1, wn2eff, bn2eff, nblk):
    n = x.shape[0]
    grid = (n // nblk,)
    blk = lambda shape: pl.BlockSpec(shape, lambda i: (0,) * len(shape))
    nodeb = pl.BlockSpec((nblk, H), lambda i: (i, 0))
    inc0 = pl.BlockSpec((1, nblk, H), lambda i: (0, i, 0))
    inc1 = pl.BlockSpec((1, nblk, H), lambda i: (1, i, 0))
    return pl.pallas_call(
        _node_apply_body,
        grid=grid,
        in_specs=[nodeb, inc0, inc1, nodeb, blk((1, H)), blk((H, H)),
                  blk((1, H)), blk((H, H)), blk((1, H))],
        out_specs=nodeb,
        out_shape=jax.ShapeDtypeStruct((n, H), F32),
    )(xd, inc, inc, x, v, wn1, bn1, wn2eff, bn2eff)


def kernel(x, edge_attr, edge_index, Wb, bb, w_e1, b_e1, g_e1, bt_e1,
           w_e2, b_e2, g_e2, bt_e2, w_e3, b_e3, w_d, b_d, v,
           w_n1, b_n1, g_n1, bt_n1, w_n2, b_n2):
    n = x.shape[0]
    e = edge_attr.shape[0]

    w1a = w_e1[0:H]
    wmid = w_e1[H:2 * H]
    w1c = w_e1[2 * H:3 * H]

    row = lambda b: b.reshape(1, -1)

    rmat = jnp.repeat(jnp.eye(4, dtype=F32) * NSTEP, NSTEP, axis=1)
    cf = (jnp.tile(jnp.linspace(0.0, 1.0, NSTEP), 4) * NSTEP).astype(F32).reshape(1, -1)

    tsrc, u3, xd = _prep(x, w1a, w1c, w_d, row(b_d), nblk=2000)

    eat = edge_attr.T
    one18 = jnp.zeros((1, 8), F32).at[0, 0].set(1.0)

    eb = 2560
    bounds = [0, 81920, 163840, 245760, e]
    parts = [(bounds[i], bounds[i + 1] - bounds[i]) for i in range(4)]
    wbb = Wb.astype(BF16)
    wmidb = wmid.astype(BF16)

    gres = [_sc_gather(tsrc, u3, edge_index, e0, sz) for e0, sz in parts]
    p1 = [_pass1(eat, gs_, gd_, rmat, cf, wbb, row(bb), wmidb, row(b_e1),
                 eb, e0 // eb)
          for (e0, sz), (gs_, gd_) in zip(parts, gres)]
    st1 = sum(st_ for _, st_ in p1)
    w2eff, b2eff = _bn_fold(st1, e, g_e1, bt_e1, w_e2, b_e2)
    w2b = w2eff.astype(BF16)
    st2 = sum(_stats2(h1_, w2b, b2eff, eb) for h1_, _ in p1)
    w3eff, b3eff = _bn_fold(st2, e, g_e2, bt_e2, w_e3, b_e3)
    w3b = w3eff.astype(BF16)
    ms = [_msgs(h1_, gs_, eat, one18, w2b, b2eff, w3b, b3eff, eb, e0 // eb)
          for (e0, sz), (h1_, _), (gs_, _) in zip(parts, p1, gres)]

    inc = jnp.zeros((SC_CORES, n, H), F32)
    for (e0, sz), m_ in zip(parts, ms):
        inc = _sc_scatter(m_, edge_index, inc, e0)

    stn = _node_stats(xd, inc, v, w_n1, row(b_n1), nblk=2000)
    wn2eff, bn2eff = _bn_fold(stn, n, g_n1, bt_n1, w_n2, b_n2)
    return _node_apply(xd, inc, x, v, w_n1, row(b_n1), wn2eff, bn2eff,
                       nblk=2000)


# speedup vs baseline: 1.0460x; 1.0460x over previous
import functools

import numpy as np
import jax
import jax.numpy as jnp
from jax import lax
from jax.experimental import pallas as pl
from jax.experimental.pallas import tpu as pltpu
from jax.experimental.pallas import tpu_sc as plsc

F32 = jnp.float32
BF16 = jnp.bfloat16
H = 128
NSTEP = 50
EPS = 1e-5

SC_CORES = 2
SC_SUBCORES = 16
SC_WORKERS = SC_CORES * SC_SUBCORES


def _lrelu(x):
    return jnp.where(x >= 0, x, 0.01 * x)


_MASK_HI = -65536
_MASK_LO = 0xFFFF


def _rne_bf16_bits(f):
    b = lax.bitcast_convert_type(f, jnp.int32)
    return b + 0x7FFF + ((b >> 16) & 1)


def _pack_pair(lo, hi):
    return (((_rne_bf16_bits(lo) >> 16) & _MASK_LO)
            | (_rne_bf16_bits(hi) & _MASK_HI))


def _unpack_lo(p):
    return lax.bitcast_convert_type(p << 16, F32)


def _unpack_hi(p):
    return lax.bitcast_convert_type(p & _MASK_HI, F32)


def _dot16(a, b):
    return jnp.dot(a.astype(BF16), b.astype(BF16), preferred_element_type=F32)


def _prep_body(x_ref, w1a_ref, w1c_ref, wd_ref, bd_ref, tsrc_ref, u3_ref,
               xd_ref):
    xb = x_ref[...]
    xd = jnp.dot(xb, wd_ref[...], preferred_element_type=F32) + bd_ref[...]
    u1 = jnp.dot(xb, w1a_ref[...], preferred_element_type=F32)
    tsrc_ref[...] = _pack_pair(u1, xd)
    u3_ref[...] = jnp.dot(xb, w1c_ref[...], preferred_element_type=F32)
    xd_ref[...] = xd


def _prep(x, w1a, w1c, wd, bd, nblk):
    n = x.shape[0]
    grid = (n // nblk,)
    return pl.pallas_call(
        _prep_body,
        grid=grid,
        in_specs=[
            pl.BlockSpec((nblk, H), lambda i: (i, 0)),
            pl.BlockSpec((H, H), lambda i: (0, 0)),
            pl.BlockSpec((H, H), lambda i: (0, 0)),
            pl.BlockSpec((H, H), lambda i: (0, 0)),
            pl.BlockSpec((1, H), lambda i: (0, 0)),
        ],
        out_specs=[
            pl.BlockSpec((nblk, H), lambda i: (i, 0)),
            pl.BlockSpec((nblk, H), lambda i: (i, 0)),
            pl.BlockSpec((nblk, H), lambda i: (i, 0)),
        ],
        out_shape=[
            jax.ShapeDtypeStruct((n, H), jnp.int32),
            jax.ShapeDtypeStruct((n, H), F32),
            jax.ShapeDtypeStruct((n, H), F32),
        ],
    )(x, w1a, w1c, wd, bd)


def _sc_gather(tsrc, u3, ei, e0, e):
    tc = e // 128
    cw = tc // SC_WORKERS
    rem = tc - cw * SC_WORKERS
    assert e % 128 == 0 and rem % 2 == 0
    stage = (cw + rem) * 128
    mesh = plsc.VectorSubcoreMesh(core_axis_name="c", subcore_axis_name="s")

    @functools.partial(
        pl.kernel,
        mesh=mesh,
        out_type=[
            jax.ShapeDtypeStruct((e, H), jnp.int32),
            jax.ShapeDtypeStruct((e, H), F32),
        ],
        scratch_types=[
            pltpu.VMEM((stage,), jnp.int32),
            pltpu.VMEM((stage,), jnp.int32),
            pltpu.VMEM((128, H), jnp.int32),
            pltpu.VMEM((128, H), jnp.int32),
            pltpu.VMEM((128, H), F32),
            pltpu.VMEM((128, H), F32),
            pltpu.SemaphoreType.DMA,
            pltpu.SemaphoreType.DMA,
            pltpu.SemaphoreType.DMA,
            pltpu.SemaphoreType.DMA,
        ],
    )
    def k(tsrc_hbm, u3_hbm, ei_hbm, gs_hbm, gd_hbm,
          idxs, idxd, rs0, rs1, rd0, rd1, sem_s0, sem_s1, sem_d0, sem_d1):
        c = lax.axis_index("c")
        s = lax.axis_index("s")
        wid = s * SC_CORES + c
        base = wid * cw * 128
        nf = cw + jnp.where(wid == SC_WORKERS - 1, rem, 0)

        pltpu.sync_copy(ei_hbm.at[0, pl.ds(e0 + base, stage)], idxs)
        pltpu.sync_copy(ei_hbm.at[1, pl.ds(e0 + base, stage)], idxd)

        def start(j, rbuf_s, rbuf_d, sem_a, sem_b):
            pltpu.async_copy(tsrc_hbm.at[idxs.at[pl.ds(j * 128, 128)]], rbuf_s, sem_a)
            pltpu.async_copy(u3_hbm.at[idxd.at[pl.ds(j * 128, 128)]], rbuf_d, sem_b)

        def finish(j, rbuf_s, rbuf_d, sem_a, sem_b):
            pltpu.make_async_copy(tsrc_hbm.at[idxs.at[pl.ds(j * 128, 128)]], rbuf_s, sem_a).wait()
            pltpu.make_async_copy(u3_hbm.at[idxd.at[pl.ds(j * 128, 128)]], rbuf_d, sem_b).wait()
            pltpu.sync_copy(rbuf_s, gs_hbm.at[pl.ds(base + j * 128, 128)])
            pltpu.sync_copy(rbuf_d, gd_hbm.at[pl.ds(base + j * 128, 128)])

        start(0, rs0, rd0, sem_s0, sem_d0)

        def body(j2, carry):
            j = j2 * 2
            start(j + 1, rs1, rd1, sem_s1, sem_d1)
            finish(j, rs0, rd0, sem_s0, sem_d0)

            @pl.when(j + 2 < nf)
            def _():
                start(j + 2, rs0, rd0, sem_s0, sem_d0)

            finish(j + 1, rs1, rd1, sem_s1, sem_d1)
            return carry

        lax.fori_loop(0, nf // 2, body, 0)
        if cw % 2:
            finish(nf - 1, rs0, rd0, sem_s0, sem_d0)

    return k(tsrc, u3, ei)


_DN0 = (((0,), (0,)), ((), ()))


def _pass1_body(ea_ref, gs_ref, gd_ref, r_ref, cf_ref, wb_ref, bb_ref,
                wmid_ref, be1_ref, h1_ref, st_ref, acc):
    i = pl.program_id(0)
    a = lax.dot_general(ea_ref[...], r_ref[...], _DN0,
                        preferred_element_type=F32)
    d = a - cf_ref[...]
    g = jnp.exp(-(d * d))
    g = _lrelu(_dot16(g, wb_ref[...]) + bb_ref[...])
    h1 = _lrelu(_unpack_lo(gs_ref[...]) + gd_ref[...] +
                _dot16(g, wmid_ref[...]) + be1_ref[...])
    h1_ref[...] = h1.astype(BF16)

    @pl.when(i == 0)
    def _():
        acc[...] = jnp.zeros_like(acc)

    acc[0:1, :] += jnp.sum(h1, axis=0, keepdims=True)
    acc[1:2, :] += jnp.sum(h1 * h1, axis=0, keepdims=True)

    @pl.when(i == pl.num_programs(0) - 1)
    def _():
        st_ref[...] = acc[...]


def _pass1(ea, gs, gd, rmat, cf, wb, bb, wmid, be1, eb, blk0):
    e = gs.shape[0]
    grid = (e // eb,)
    return pl.pallas_call(
        _pass1_body,
        grid=grid,
        in_specs=[
            pl.BlockSpec((4, eb), lambda i: (0, blk0 + i)),
            pl.BlockSpec((eb, H), lambda i: (i, 0)),
            pl.BlockSpec((eb, H), lambda i: (i, 0)),
            pl.BlockSpec((4, 4 * NSTEP), lambda i: (0, 0)),
            pl.BlockSpec((1, 4 * NSTEP), lambda i: (0, 0)),
            pl.BlockSpec((4 * NSTEP, H), lambda i: (0, 0)),
            pl.BlockSpec((1, H), lambda i: (0, 0)),
            pl.BlockSpec((H, H), lambda i: (0, 0)),
            pl.BlockSpec((1, H), lambda i: (0, 0)),
        ],
        out_specs=[
            pl.BlockSpec((eb, H), lambda i: (i, 0)),
            pl.BlockSpec((8, H), lambda i: (0, 0)),
        ],
        out_shape=[
            jax.ShapeDtypeStruct((e, H), BF16),
            jax.ShapeDtypeStruct((8, H), F32),
        ],
        scratch_shapes=[pltpu.VMEM((8, H), F32)],
    )(ea, gs, gd, rmat, cf, wb, bb, wmid, be1)


def _bn_fold(st, ne, g, bt, w_next, b_next):
    mean = st[0] * (1.0 / ne)
    var = st[1] * (1.0 / ne) - mean * mean
    a = g * lax.rsqrt(var + EPS)
    weff = w_next * a[:, None]
    beff = ((bt - mean * a) @ w_next + b_next).reshape(1, -1)
    return weff, beff


def _stats2_body(h1_ref, w2_ref, b2_ref, st2_ref, acc):
    i = pl.program_id(0)
    h2 = _lrelu(_dot16(h1_ref[...], w2_ref[...]) + b2_ref[...])

    @pl.when(i == 0)
    def _():
        acc[...] = jnp.zeros_like(acc)

    acc[0:1, :] += jnp.sum(h2, axis=0, keepdims=True)
    acc[1:2, :] += jnp.sum(h2 * h2, axis=0, keepdims=True)

    @pl.when(i == pl.num_programs(0) - 1)
    def _():
        st2_ref[...] = acc[...]


def _stats2(h1, w2eff, b2eff, eb):
    e = h1.shape[0]
    grid = (e // eb,)
    return pl.pallas_call(
        _stats2_body,
        grid=grid,
        in_specs=[
            pl.BlockSpec((eb, H), lambda i: (i, 0)),
            pl.BlockSpec((H, H), lambda i: (0, 0)),
            pl.BlockSpec((1, H), lambda i: (0, 0)),
        ],
        out_specs=pl.BlockSpec((8, H), lambda i: (0, 0)),
        out_shape=jax.ShapeDtypeStruct((8, H), F32),
        scratch_shapes=[pltpu.VMEM((8, H), F32)],
    )(h1, w2eff, b2eff)


def _msgs_body(h1_ref, gxd_ref, ea_ref, one_ref, w2_ref, b2_ref,
               w3_ref, b3_ref, m_ref):
    h2 = _lrelu(_dot16(h1_ref[...], w2_ref[...]) + b2_ref[...])
    h3 = _dot16(h2, w3_ref[...]) + b3_ref[...]
    cosr = jnp.cos((np.pi / 2) * ea_ref[3:4, :])
    coef = lax.dot_general(cosr, one_ref[...], _DN0,
                           preferred_element_type=F32)
    m_ref[...] = coef[:, 0:1] * h3 * _unpack_hi(gxd_ref[...])


def _msgs(h1, gs, eat, one18, w2eff, b2eff, w3eff, b3eff, eb, blk0):
    e = h1.shape[0]
    grid = (e // eb,)
    return pl.pallas_call(
        _msgs_body,
        grid=grid,
        in_specs=[
            pl.BlockSpec((eb, H), lambda i: (i, 0)),
            pl.BlockSpec((eb, H), lambda i: (i, 0)),
            pl.BlockSpec((4, eb), lambda i: (0, blk0 + i)),
            pl.BlockSpec((1, 8), lambda i: (0, 0)),
            pl.BlockSpec((H, H), lambda i: (0, 0)),
            pl.BlockSpec((1, H), lambda i: (0, 0)),
            pl.BlockSpec((H, H), lambda i: (0, 0)),
            pl.BlockSpec((1, H), lambda i: (0, 0)),
        ],
        out_specs=pl.BlockSpec((eb, H), lambda i: (i, 0)),
        out_shape=jax.ShapeDtypeStruct((e, H), F32),
    )(h1, gs, eat, one18, w2eff, b2eff, w3eff, b3eff)


def _sc_scatter(m, ei, init, e0):
    e = m.shape[0]
    n = init.shape[1]
    tc = e // 128
    cw = tc // SC_WORKERS
    rem = tc - cw * SC_WORKERS
    assert e % 128 == 0 and rem % 2 == 0
    mesh = plsc.VectorSubcoreMesh(core_axis_name="c", subcore_axis_name="s")

    @functools.partial(
        pl.kernel,
        mesh=mesh,
        out_type=jax.ShapeDtypeStruct((SC_CORES, n, H), F32),
        scratch_types=[
            pltpu.VMEM((128,), jnp.int32),
            pltpu.VMEM((128,), jnp.int32),
            pltpu.VMEM((16,), jnp.int32),
            pltpu.VMEM((128, H), F32),
            pltpu.VMEM((128, H), F32),
            pltpu.VMEM_SHARED((n, H), F32),
            pltpu.SemaphoreType.DMA,
            pltpu.SemaphoreType.DMA,
            pltpu.SemaphoreType.DMA,
            pltpu.SemaphoreType.DMA,
        ],
    )
    def k(m_hbm, ei_hbm, z_hbm, out_hbm, idx0, idx1, idxt, rb0, rb1, accsh,
          si0, si1, sm0, sm1):
        c = lax.axis_index("c")
        s = lax.axis_index("s")
        r0 = s * 640
        ncp = jnp.where(s == SC_SUBCORES - 1, (n - 640 * (SC_SUBCORES - 1)) // 40,
                        640 // 40)

        def cp_init(j, carry):
            off = r0 + j * 40
            pltpu.sync_copy(z_hbm.at[c, pl.ds(off, 40)], accsh.at[pl.ds(off, 40)])
            return carry

        lax.fori_loop(0, ncp, cp_init, 0)
        wid = c * SC_SUBCORES + s
        base = wid * cw * 128
        nf = cw + jnp.where(wid == SC_WORKERS - 1, rem, 0)
        plsc.subcore_barrier()

        def start(j, idxb, rbuf, semi, semm):
            pltpu.async_copy(ei_hbm.at[1, pl.ds(e0 + base + j * 128, 128)], idxb, semi)
            pltpu.async_copy(m_hbm.at[pl.ds(base + j * 128, 128)], rbuf, semm)

        def finish(j, idxb, rbuf, semi, semm):
            pltpu.make_async_copy(ei_hbm.at[1, pl.ds(e0 + base + j * 128, 128)], idxb, semi).wait()
            pltpu.make_async_copy(m_hbm.at[pl.ds(base + j * 128, 128)], rbuf, semm).wait()
            pltpu.sync_copy(rbuf, accsh.at[idxb], add=True)

        start(0, idx0, rb0, si0, sm0)

        def body(j2, carry):
            j = j2 * 2
            start(j + 1, idx1, rb1, si1, sm1)
            finish(j, idx0, rb0, si0, sm0)

            @pl.when(j + 2 < nf)
            def _():
                start(j + 2, idx0, rb0, si0, sm0)

            finish(j + 1, idx1, rb1, si1, sm1)
            return carry

        lax.fori_loop(0, nf // 2, body, 0)
        if cw % 2:
            finish(nf - 1, idx0, rb0, si0, sm0)
        plsc.subcore_barrier()

        def cp_out(j, carry):
            off = r0 + j * 40
            pltpu.sync_copy(accsh.at[pl.ds(off, 40)], out_hbm.at[c, pl.ds(off, 40)])
            return carry

        lax.fori_loop(0, ncp, cp_out, 0)

    return k(m, ei, init)


def _node_stats_body(xd_ref, i0_ref, i1_ref, v_ref, wn1_ref, bn1_ref,
                     st_ref, acc):
    i = pl.program_id(0)
    z0 = v_ref[...] * xd_ref[...] + i0_ref[0] + i1_ref[0]
    n1 = _lrelu(jnp.dot(z0, wn1_ref[...], preferred_element_type=F32) + bn1_ref[...])

    @pl.when(i == 0)
    def _():
        acc[...] = jnp.zeros_like(acc)

    acc[0:1, :] += jnp.sum(n1, axis=0, keepdims=True)
    acc[1:2, :] += jnp.sum(n1 * n1, axis=0, keepdims=True)

    @pl.when(i == pl.num_programs(0) - 1)
    def _():
        st_ref[...] = acc[...]


def _node_apply_body(xd_ref, i0_ref, i1_ref, x_ref, v_ref, wn1_ref,
                     bn1_ref, wn2_ref, bn2_ref, z_ref):
    z0 = v_ref[...] * xd_ref[...] + i0_ref[0] + i1_ref[0]
    n1 = _lrelu(jnp.dot(z0, wn1_ref[...], preferred_element_type=F32) + bn1_ref[...])
    z_ref[...] = (jnp.dot(n1, wn2_ref[...], preferred_element_type=F32)
                  + bn2_ref[...] + x_ref[...])


def _node_stats(xd, inc, v, wn1, bn1, nblk):
    n = xd.shape[0]
    grid = (n // nblk,)
    blk = lambda shape: pl.BlockSpec(shape, lambda i: (0,) * len(shape))
    nodeb = pl.BlockSpec((nblk, H), lambda i: (i, 0))
    inc0 = pl.BlockSpec((1, nblk, H), lambda i: (0, i, 0))
    inc1 = pl.BlockSpec((1, nblk, H), lambda i: (1, i, 0))
    return pl.pallas_call(
        _node_stats_body,
        grid=grid,
        in_specs=[nodeb, inc0, inc1, blk((1, H)), blk((H, H)), blk((1, H))],
        out_specs=pl.BlockSpec((8, H), lambda i: (0, 0)),
        out_shape=jax.ShapeDtypeStruct((8, H), F32),
        scratch_shapes=[pltpu.VMEM((8, H), F32)],
    )(xd, inc, inc, v, wn1, bn1)


def _node_apply(xd, inc, x, v, wn1, bn1, wn2eff, bn2eff, nblk):
    n = x.shape[0]
    grid = (n // nblk,)
    blk = lambda shape: pl.BlockSpec(shape, lambda i: (0,) * len(shape))
    nodeb = pl.BlockSpec((nblk, H), lambda i: (i, 0))
    inc0 = pl.BlockSpec((1, nblk, H), lambda i: (0, i, 0))
    inc1 = pl.BlockSpec((1, nblk, H), lambda i: (1, i, 0))
    return pl.pallas_call(
        _node_apply_body,
        grid=grid,
        in_specs=[nodeb, inc0, inc1, nodeb, blk((1, H)), blk((H, H)),
                  blk((1, H)), blk((H, H)), blk((1, H))],
        out_specs=nodeb,
        out_shape=jax.ShapeDtypeStruct((n, H), F32),
    )(xd, inc, inc, x, v, wn1, bn1, wn2eff, bn2eff)


def kernel(x, edge_attr, edge_index, Wb, bb, w_e1, b_e1, g_e1, bt_e1,
           w_e2, b_e2, g_e2, bt_e2, w_e3, b_e3, w_d, b_d, v,
           w_n1, b_n1, g_n1, bt_n1, w_n2, b_n2):
    n = x.shape[0]
    e = edge_attr.shape[0]

    w1a = w_e1[0:H]
    wmid = w_e1[H:2 * H]
    w1c = w_e1[2 * H:3 * H]

    row = lambda b: b.reshape(1, -1)

    rmat = jnp.repeat(jnp.eye(4, dtype=F32) * NSTEP, NSTEP, axis=1)
    cf = (jnp.tile(jnp.linspace(0.0, 1.0, NSTEP), 4) * NSTEP).astype(F32).reshape(1, -1)

    tsrc, u3, xd = _prep(x, w1a, w1c, w_d, row(b_d), nblk=2000)

    eat = edge_attr.T
    one18 = jnp.zeros((1, 8), F32).at[0, 0].set(1.0)

    eb = 3200
    bounds = [0, e // 2, e]
    parts = [(bounds[i], bounds[i + 1] - bounds[i]) for i in range(len(bounds) - 1)]
    wbb = Wb.astype(BF16)
    wmidb = wmid.astype(BF16)

    gres = [_sc_gather(tsrc, u3, edge_index, e0, sz) for e0, sz in parts]
    p1 = [_pass1(eat, gs_, gd_, rmat, cf, wbb, row(bb), wmidb, row(b_e1),
                 eb, e0 // eb)
          for (e0, sz), (gs_, gd_) in zip(parts, gres)]
    st1 = sum(st_ for _, st_ in p1)
    w2eff, b2eff = _bn_fold(st1, e, g_e1, bt_e1, w_e2, b_e2)
    w2b = w2eff.astype(BF16)
    st2 = sum(_stats2(h1_, w2b, b2eff, eb) for h1_, _ in p1)
    w3eff, b3eff = _bn_fold(st2, e, g_e2, bt_e2, w_e3, b_e3)
    w3b = w3eff.astype(BF16)
    ms = [_msgs(h1_, gs_, eat, one18, w2b, b2eff, w3b, b3eff, eb, e0 // eb)
          for (e0, sz), (h1_, _), (gs_, _) in zip(parts, p1, gres)]

    inc = jnp.zeros((SC_CORES, n, H), F32)
    for (e0, sz), m_ in zip(parts, ms):
        inc = _sc_scatter(m_, edge_index, inc, e0)

    stn = _node_stats(xd, inc, v, w_n1, row(b_n1), nblk=2000)
    wn2eff, bn2eff = _bn_fold(stn, n, g_n1, bt_n1, w_n2, b_n2)
    return _node_apply(xd, inc, x, v, w_n1, row(b_n1), wn2eff, bn2eff,
                       nblk=2000)
